# Initial kernel scaffold; baseline (speedup 1.0000x reference)
import jax
import jax.numpy as jnp
from jax.experimental import pallas as pl


def kernel(x):
    raise NotImplementedError("write your pallas kernel here")


# speedup vs baseline: 6.8112x; 6.8112x over previous
import functools

import jax
import jax.numpy as jnp
from jax import lax
from jax.experimental import pallas as pl
from jax.experimental.pallas import tpu as pltpu
from jax.experimental.pallas import tpu_sc as plsc

B = 16384
T = 200
S = 21
N = B * T

NUM_CORES = 2
NUM_SUBCORES = 16
NW = NUM_CORES * NUM_SUBCORES
PER_W = N // NW
C = 2048
ITERS = PER_W // C
VECS = C // 16

_mesh = plsc.VectorSubcoreMesh(core_axis_name="c", subcore_axis_name="s")


@functools.partial(
    pl.kernel,
    mesh=_mesh,
    out_type=jax.ShapeDtypeStruct((N * S,), jnp.float32),
    scratch_types=[
        pltpu.VMEM((C,), jnp.float32),
        pltpu.VMEM((C * S,), jnp.float32),
    ],
    compiler_params=pltpu.CompilerParams(needs_layout_passes=False),
)
def _two_hot(x_hbm, out_hbm, x_v, out_v):
    wid = lax.axis_index("s") * NUM_CORES + lax.axis_index("c")
    base = wid * PER_W

    iota = lax.iota(jnp.int32, 16)
    zeros = jnp.zeros((16,), jnp.float32)

    def chunk_body(g, carry):
        off = base + g * C
        pltpu.sync_copy(x_hbm.at[pl.ds(off, C)], x_v)

        def elem_body(i, c2):
            xv = x_v[pl.ds(i * 16, 16)]
            xc = jnp.minimum(jnp.maximum(xv, 0.0), 20.0)
            lo = xc.astype(jnp.int32)
            frac = xc - lo.astype(jnp.float32)
            hi = jnp.minimum(lo + 1, S - 1)
            row0 = i * (16 * S)
            for k in range(S):
                out_v[pl.ds(row0 + k * 16, 16)] = zeros
            pos = (i * 16 + iota) * S
            plsc.store_scatter(out_v, [pos + hi], frac)
            plsc.store_scatter(out_v, [pos + lo], 1.0 - frac)
            return c2

        lax.fori_loop(0, VECS, elem_body, 0)
        pltpu.sync_copy(out_v, out_hbm.at[pl.ds(off * S, C * S)])
        return carry

    lax.fori_loop(0, ITERS, chunk_body, 0)


def kernel(x):
    out = _two_hot(x.reshape(N))
    return out.reshape(B, T, S)


# speedup vs baseline: 114.0650x; 16.7466x over previous
import functools

import jax
import jax.numpy as jnp
from jax import lax
from jax.experimental import pallas as pl
from jax.experimental.pallas import tpu as pltpu
from jax.experimental.pallas import tpu_sc as plsc

B = 16384
T = 200
S = 21
N = B * T
TR = T // 8
BC = B // 128

NUM_CORES = 2
NUM_SUBCORES = 16
NW = NUM_CORES * NUM_SUBCORES
BPW = B // NW
NQ = BPW // 128
XQ = 128 * T

_mesh = plsc.VectorSubcoreMesh(core_axis_name="c", subcore_axis_name="s")


@functools.partial(
    pl.kernel,
    mesh=_mesh,
    out_type=jax.ShapeDtypeStruct((S, TR, BC, 8, 128), jnp.float32),
    scratch_types=[
        pltpu.VMEM((XQ,), jnp.float32),
        pltpu.VMEM((S, 1, 1, 8, 128), jnp.float32),
        pltpu.VMEM((S, 1, 1, 8, 128), jnp.float32),
        pltpu.SemaphoreType.DMA,
        pltpu.SemaphoreType.DMA,
    ],
    compiler_params=pltpu.CompilerParams(needs_layout_passes=False),
)
def _two_hot(x_hbm, out_hbm, x_q, buf0, buf1, sem0, sem1):
    wid = lax.axis_index("s") * NUM_CORES + lax.axis_index("c")

    iota = lax.iota(jnp.int32, 16)
    brelv = lax.shift_right_logical(iota, 3)
    t8v = lax.bitwise_and(iota, 7)
    zeros = jnp.zeros((16,), jnp.float32)

    def process(cp, q, tr, buf, sem):
        b0 = wid * BPW + q * 128
        bc = wid * NQ + q

        @pl.when(tr == 0)
        def _load_x():
            pltpu.sync_copy(x_hbm.at[pl.ds(b0 * T, XQ)], x_q)

        @pl.when(cp > 0)
        def _drain():
            pltpu.make_async_copy(
                buf, out_hbm.at[:, pl.ds(tr, 1), pl.ds(bc, 1)], sem
            ).wait()

        def zero_k(k, c):
            for t8 in range(8):
                for l in range(8):
                    buf[k, 0, 0, t8, pl.ds(l * 16, 16)] = zeros
            return c

        lax.fori_loop(0, S, zero_k, 0)

        tr8 = tr * 8

        def compute(i, c):
            brel = brelv + i * 2
            xidx = brel * T + (tr8 + t8v)
            xv = plsc.load_gather(x_q, [xidx])
            xc = jnp.minimum(jnp.maximum(xv, 0.0), 20.0)
            lo = xc.astype(jnp.int32)
            frac = xc - lo.astype(jnp.float32)
            hi = jnp.minimum(lo + 1, S - 1)
            z0 = jnp.zeros((16,), jnp.int32)
            plsc.store_scatter(buf, [hi, z0, z0, t8v, brel], frac)
            plsc.store_scatter(buf, [lo, z0, z0, t8v, brel], 1.0 - frac)
            return c

        lax.fori_loop(0, 64, compute, 0)

        pltpu.async_copy(buf, out_hbm.at[:, pl.ds(tr, 1), pl.ds(bc, 1)], sem)

    def incr(q, tr):
        last = tr == TR - 1
        return jnp.where(last, q + 1, q), jnp.where(last, 0, tr + 1)

    def pair_body(cp, qt):
        q, tr = qt
        process(cp, q, tr, buf0, sem0)
        q, tr = incr(q, tr)
        process(cp, q, tr, buf1, sem1)
        return incr(q, tr)

    lax.fori_loop(0, NQ * TR // 2, pair_body, (jnp.int32(0), jnp.int32(0)))

    for buf, sem in ((buf0, sem0), (buf1, sem1)):
        pltpu.make_async_copy(
            buf, out_hbm.at[:, pl.ds(0, 1), pl.ds(0, 1)], sem
        ).wait()


def kernel(x):
    out5 = _two_hot(x.reshape(N))
    return out5.transpose(2, 4, 1, 3, 0).reshape(B, T, S)


# speedup vs baseline: 156.7955x; 1.3746x over previous
import functools

import jax
import jax.numpy as jnp
from jax import lax
from jax.experimental import pallas as pl
from jax.experimental.pallas import tpu as pltpu
from jax.experimental.pallas import tpu_sc as plsc

B = 16384
T = 200
S = 21
N = B * T
TR = T // 8
BC = B // 128

NUM_CORES = 2
NUM_SUBCORES = 16
NW = NUM_CORES * NUM_SUBCORES
NQ = BC // NW
NCHUNK = NQ * TR
NBUF = 5

_mesh = plsc.VectorSubcoreMesh(core_axis_name="c", subcore_axis_name="s")

_OUT_BUFS = [pltpu.VMEM((S, 1, 1, 8, 128), jnp.float32) for _ in range(NBUF)]
_X_BUFS = [pltpu.VMEM((1, 1, 8, 128), jnp.float32) for _ in range(NBUF)]
_SEMS = [pltpu.SemaphoreType.DMA for _ in range(2 * NBUF)]


@functools.partial(
    pl.kernel,
    mesh=_mesh,
    out_type=jax.ShapeDtypeStruct((S, TR, BC, 8, 128), jnp.float32),
    scratch_types=_OUT_BUFS + _X_BUFS + _SEMS,
    compiler_params=pltpu.CompilerParams(needs_layout_passes=False),
)
def _two_hot(x_hbm, out_hbm, *scratch):
    bufs = scratch[:NBUF]
    xbufs = scratch[NBUF : 2 * NBUF]
    osems = scratch[2 * NBUF : 3 * NBUF]
    xsems = scratch[3 * NBUF :]
    wid = lax.axis_index("s") * NUM_CORES + lax.axis_index("c")
    bc0 = wid * NQ

    iota = lax.iota(jnp.int32, 16)
    zeros = jnp.zeros((16,), jnp.float32)
    z0 = jnp.zeros((16,), jnp.int32)

    def x_slice(q, tr):
        return x_hbm.at[pl.ds(tr, 1), pl.ds(bc0 + q, 1)]

    def out_slice(q, tr):
        return out_hbm.at[:, pl.ds(tr, 1), pl.ds(bc0 + q, 1)]

    for sub in range(NBUF):
        pltpu.async_copy(x_slice(0, sub), xbufs[sub], xsems[sub])

    def process(cp, q, tr, sub):
        buf, xb = bufs[sub], xbufs[sub]

        pltpu.make_async_copy(x_slice(q, tr), xb, xsems[sub]).wait()

        @pl.when(cp > 0)
        def _drain():
            pltpu.make_async_copy(buf, out_slice(q, tr), osems[sub]).wait()

        def zero_k(k, c):
            for t8 in range(8):
                for l in range(8):
                    buf[k, 0, 0, t8, pl.ds(l * 16, 16)] = zeros
            return c

        lax.fori_loop(0, S, zero_k, 0)

        def compute(i, c):
            t8s = lax.shift_right_logical(i, 3)
            c16 = lax.bitwise_and(i, 7) * 16
            b128 = c16 + iota
            xv = xb[0, 0, t8s, pl.ds(c16, 16)]
            xc = jnp.minimum(jnp.maximum(xv, 0.0), 20.0)
            lo = xc.astype(jnp.int32)
            frac = xc - lo.astype(jnp.float32)
            hi = jnp.minimum(lo + 1, S - 1)
            t8v = jnp.full((16,), t8s, jnp.int32)
            plsc.store_scatter(buf, [hi, z0, z0, t8v, b128], frac)
            plsc.store_scatter(buf, [lo, z0, z0, t8v, b128], 1.0 - frac)
            return c

        lax.fori_loop(0, 64, compute, 0)

        pltpu.async_copy(buf, out_slice(q, tr), osems[sub])

        wrap = tr + NBUF >= TR
        q_pf = jnp.where(wrap, q + 1, q)
        tr_pf = jnp.where(wrap, tr + NBUF - TR, tr + NBUF)

        @pl.when(q_pf < NQ)
        def _prefetch():
            pltpu.async_copy(x_slice(q_pf, tr_pf), xb, xsems[sub])

    def incr(q, tr):
        last = tr == TR - 1
        return jnp.where(last, q + 1, q), jnp.where(last, 0, tr + 1)

    def group_body(cp, qt):
        q, tr = qt
        for sub in range(NBUF):
            process(cp, q, tr, sub)
            q, tr = incr(q, tr)
        return q, tr

    lax.fori_loop(0, NCHUNK // NBUF, group_body, (jnp.int32(0), jnp.int32(0)))

    for sub in range(NBUF):
        pltpu.make_async_copy(
            bufs[sub], out_hbm.at[:, pl.ds(0, 1), pl.ds(0, 1)], osems[sub]
        ).wait()


def kernel(x):
    xp = x.reshape(BC, 128, TR, 8).transpose(2, 0, 3, 1)
    out5 = _two_hot(xp)
    return out5.transpose(2, 4, 1, 3, 0).reshape(B, T, S)
